# baseline (speedup 1.0000x reference)
import functools

import jax
import jax.numpy as jnp
from jax import lax
from jax.experimental import pallas as pl
from jax.experimental.pallas import tpu as pltpu
from jax.experimental.pallas import tpu_sc as plsc

_K = 64
_M = 128
_N = 32768
_L = 16
_NVEC = _N // _L
_NC = 2
_NW = 32
_TC_ROWS = 32
_SC_ROWS = _M - _TC_ROWS
_ROWS_PER_W = _SC_ROWS // _NW
_B25 = 0x40200000
_B10 = 0x3F800000


def _splat_total(acc):
    return plsc.cummax(lax.rev(plsc.cumsum(acc), (0,)))


def _splat_max(acc):
    return plsc.cummax(lax.rev(plsc.cummax(acc), (0,)))


def _splat_to_scalar(s, nbits=16):
    out = jnp.int32(0)
    for b in range(nbits):
        bit = jnp.any((s & (1 << b)) != 0)
        out = out + jnp.where(bit, jnp.int32(1 << b), jnp.int32(0))
    return out


def _sc_body(x_hbm, o_hbm, row_v, cix_v, stage_v, vid_v, sem_in, sem_out):
    wid = lax.axis_index("s") * _NC + lax.axis_index("c")
    lanes = lax.iota(jnp.int32, _L)
    one = jnp.ones((_L,), jnp.int32)
    zero = jnp.zeros((_L,), jnp.int32)
    zero16f = jnp.zeros((_L,), jnp.float32)
    kvec = jnp.full((_L,), _K, jnp.int32)
    sixteen = jnp.full((_L,), _L, jnp.int32)
    lane0 = lanes < one
    ninf = jnp.full((_L,), -jnp.inf, jnp.float32)

    def zero_body(i, c):
        stage_v[pl.ds(i * _L, _L)] = zero16f
        return c

    lax.fori_loop(0, _NVEC, zero_body, 0, unroll=8)

    def vz_body(i, c):
        vid_v[pl.ds(i * _L, _L)] = zero
        return c

    lax.fori_loop(0, _NVEC // _L, vz_body, 0, unroll=8)

    pltpu.async_copy(x_hbm.at[wid * _ROWS_PER_W], row_v.at[pl.ds(0, _N)],
                     sem_in)

    def row_body(r, carry):
        nc2_prev, nvec2_prev = carry
        row = wid * _ROWS_PER_W + r
        pltpu.make_async_copy(x_hbm.at[row], row_v.at[pl.ds(0, _N)],
                              sem_in).wait()

        def pass_a(tv, track_max):
            grp = 8

            def body(g, carry):
                nv, mx = carry
                anys = []
                for k in range(grp):
                    i = g * grp + k
                    v = row_v[pl.ds(i * _L, _L)]
                    m = v >= tv
                    anys.append(
                        jnp.minimum(plsc.all_reduce_population_count(m), one))
                    if track_max:
                        mx = jnp.maximum(mx, v)
                pref = [nv]
                for k in range(1, grp):
                    pref.append(pref[k - 1] + anys[k - 1])
                for k in range(grp):
                    i = g * grp + k
                    plsc.store_scatter(vid_v, [pref[k]], zero + i, mask=lane0)
                s1 = [anys[2 * k] + anys[2 * k + 1] for k in range(grp // 2)]
                s2 = [s1[2 * k] + s1[2 * k + 1] for k in range(grp // 4)]
                return nv + s2[0] + s2[1], mx

            return lax.fori_loop(0, _NVEC // grp, body, (zero, ninf))

        def compact2(tv, nvv):
            jmax = (_splat_to_scalar(nvv, 12) + _L - 1) // _L

            def outer(j, off):
                for k in range(_L):
                    slot = j * _L + k
                    vecid = plsc.load_gather(vid_v, [zero + slot])
                    valid = (zero + slot) < nvv
                    ixv = vecid * _L + lanes
                    v = plsc.load_gather(row_v, [ixv])
                    m = (v >= tv) & valid
                    mi = jnp.where(m, one, zero)
                    tgt = off + plsc.cumsum(mi) - mi
                    plsc.store_scatter(cix_v, [tgt], ixv, mask=m)
                    off = off + plsc.all_reduce_population_count(m)
                return off

            return lax.fori_loop(0, jmax, outer, zero)

        t25 = jnp.full((_L,), 2.5, jnp.float32)
        t10 = jnp.full((_L,), 1.0, jnp.float32)
        nv_a, mx = pass_a(t25, True)
        mxbits = plsc.bitcast(_splat_max(mx), jnp.int32)
        hi0 = jnp.maximum(mxbits + one, one)

        def wait_prev(_):
            pltpu.make_async_copy(stage_v, o_hbm.at[row - 1], sem_out).wait()
            return 0

        lax.cond(r > 0, wait_prev, lambda _: 0, 0)

        def rz_body(i, c2):
            valid = (lanes + i * _L) < nc2_prev
            ix = cix_v[pl.ds(i * _L, _L)]
            plsc.store_scatter(stage_v, [ix], zero16f, mask=valid)
            return c2

        lax.fori_loop(0, nvec2_prev, rz_body, 0)

        nc2_a = compact2(t25, nv_a)

        def tier_a(_):
            return nc2_a, jnp.full((_L,), _B25, jnp.int32)

        def tier_bc(_):
            pltpu.sync_copy(x_hbm.at[row], row_v.at[pl.ds(0, _N)])
            nv_b, _ = pass_a(t10, False)
            nc2_b = compact2(t10, nv_b)

            def tier_b(_):
                return nc2_b, jnp.full((_L,), _B10, jnp.int32)

            def tier_c(_):
                pltpu.sync_copy(x_hbm.at[row], row_v.at[pl.ds(0, _N)])
                nv_c, _ = pass_a(ninf, False)
                nc2_c = compact2(ninf, nv_c)
                return nc2_c, zero

            return lax.cond(jnp.any(nc2_b >= kvec), tier_b, tier_c, 0)

        nc2, lo0 = lax.cond(jnp.any(nc2_a >= kvec), tier_a, tier_bc, 0)
        ncand = _splat_to_scalar(nc2)
        nvec2 = (ncand + _L - 1) // _L

        def count_ge(t):
            def b(i, acc):
                valid = (lanes + i * _L) < nc2
                ix = cix_v[pl.ds(i * _L, _L)]
                v = plsc.load_gather(row_v, [ix], mask=valid)
                u = plsc.bitcast(v, jnp.int32)
                return acc + jnp.where((u >= t) & valid, one, zero)

            return _splat_total(lax.fori_loop(0, nvec2, b, zero))

        def bis_cond(lohi):
            lo, hi = lohi
            return jnp.any((hi - lo) > one)

        def bis_body(lohi):
            lo, hi = lohi
            d = hi - lo
            m1 = lo + lax.shift_right_logical(d, 2)
            m2 = lo + lax.shift_right_logical(d, 1)
            m3 = m2 + lax.shift_right_logical(d, 2)

            def b(i, accs):
                a1, a2, a3 = accs
                valid = (lanes + i * _L) < nc2
                ix = cix_v[pl.ds(i * _L, _L)]
                v = plsc.load_gather(row_v, [ix], mask=valid)
                u = plsc.bitcast(v, jnp.int32)
                a1 = a1 + jnp.where((u >= m1) & valid, one, zero)
                a2 = a2 + jnp.where((u >= m2) & valid, one, zero)
                a3 = a3 + jnp.where((u >= m3) & valid, one, zero)
                return a1, a2, a3

            a1, a2, a3 = lax.fori_loop(0, nvec2, b, (zero, zero, zero))
            p1 = _splat_total(a1) >= kvec
            p2 = _splat_total(a2) >= kvec
            p3 = _splat_total(a3) >= kvec
            nlo = jnp.where(p3, m3, jnp.where(p2, m2, jnp.where(p1, m1, lo)))
            nhi = jnp.where(p3, hi, jnp.where(p2, m3, jnp.where(p1, m2, m1)))
            return nlo, nhi

        lo, _ = lax.while_loop(bis_cond, bis_body, (lo0, hi0))
        need = kvec - count_ge(lo + one)

        def sel_body(i, base):
            valid = (lanes + i * _L) < nc2
            ix = cix_v[pl.ds(i * _L, _L)]
            v = plsc.load_gather(row_v, [ix], mask=valid)
            u = plsc.bitcast(v, jnp.int32)
            gt = (u > lo) & valid
            eq = (u == lo) & valid
            pr = plsc.cumsum(jnp.where(eq, one, zero)) + base
            sel = gt | (eq & (pr <= need))
            plsc.store_scatter(stage_v, [ix], jnp.maximum(v, 0.0), mask=sel)
            return base + plsc.all_reduce_population_count(eq)

        lax.fori_loop(0, nvec2, sel_body, zero)

        pltpu.async_copy(stage_v, o_hbm.at[row], sem_out)

        def start_next(_):
            pltpu.async_copy(x_hbm.at[row + 1], row_v.at[pl.ds(0, _N)],
                             sem_in)
            return 0

        lax.cond(r < _ROWS_PER_W - 1, start_next, lambda _: 0, 0)
        return nc2, nvec2

    lax.fori_loop(0, _ROWS_PER_W, row_body, (zero, 0))
    pltpu.make_async_copy(
        stage_v, o_hbm.at[wid * _ROWS_PER_W + _ROWS_PER_W - 1], sem_out
    ).wait()


@functools.partial(
    pl.kernel,
    out_type=jax.ShapeDtypeStruct((_SC_ROWS, _N), jnp.float32),
    mesh=plsc.VectorSubcoreMesh(core_axis_name="c", subcore_axis_name="s"),
    scratch_types=[
        pltpu.VMEM((_N + _L,), jnp.float32),
        pltpu.VMEM((_N,), jnp.int32),
        pltpu.VMEM((_N,), jnp.float32),
        pltpu.VMEM((_NVEC,), jnp.int32),
        pltpu.SemaphoreType.DMA,
        pltpu.SemaphoreType.DMA,
    ],
    compiler_params=pltpu.CompilerParams(needs_layout_passes=False),
)
def _sc_topk(x_hbm, o_hbm, row_v, cix_v, stage_v, vid_v, sem_in, sem_out):
    _sc_body(x_hbm, o_hbm, row_v, cix_v, stage_v, vid_v, sem_in, sem_out)


def _tc_topk_kernel(x_ref, o_ref):
    x = x_ref[...]
    y = jnp.maximum(x, 0.0)
    u = jax.lax.bitcast_convert_type(y, jnp.int32)
    rows = y.shape[0]
    lo0 = jnp.zeros((rows, 1), jnp.int32)
    hi0 = jnp.full((rows, 1), 0x7F800001, jnp.int32)

    def body(_, carry):
        lo, hi = carry
        mid = lo + lax.shift_right_logical(hi - lo, 1)
        cnt = jnp.sum((u >= mid).astype(jnp.int32), axis=1, keepdims=True)
        pred = cnt >= _K
        return jnp.where(pred, mid, lo), jnp.where(pred, hi, mid)

    lo, _ = lax.fori_loop(0, 31, body, (lo0, hi0))
    gt = u > lo
    eq = u == lo
    kneed = _K - jnp.sum(gt.astype(jnp.int32), axis=1, keepdims=True)
    iota = lax.broadcasted_iota(jnp.int32, y.shape, 1)
    jlo0 = jnp.full((rows, 1), -1, jnp.int32)
    jhi0 = jnp.full((rows, 1), y.shape[1] - 1, jnp.int32)

    def jbody(_, carry):
        jlo, jhi = carry
        jmid = jlo + lax.shift_right_logical(jhi - jlo, 1)
        cnt = jnp.sum((eq & (iota <= jmid)).astype(jnp.int32), axis=1,
                      keepdims=True)
        pred = cnt >= kneed
        return jnp.where(pred, jlo, jmid), jnp.where(pred, jmid, jhi)

    _, jhi = lax.fori_loop(0, 15, jbody, (jlo0, jhi0))
    mask = gt | (eq & (iota <= jhi))
    o_ref[...] = jnp.where(mask, y, 0.0)


def _tc_topk(x):
    m, n = x.shape
    rows_per_block = 16
    return pl.pallas_call(
        _tc_topk_kernel,
        grid=(m // rows_per_block,),
        in_specs=[pl.BlockSpec((rows_per_block, n), lambda i: (i, 0))],
        out_specs=pl.BlockSpec((rows_per_block, n), lambda i: (i, 0)),
        out_shape=jax.ShapeDtypeStruct((m, n), x.dtype),
    )(x)


def kernel(x):
    sc_out = _sc_topk(x[_TC_ROWS:])
    tc_out = _tc_topk(x[:_TC_ROWS])
    return jnp.concatenate([tc_out, sc_out], axis=0)


# speedup vs baseline: 1.1279x; 1.1279x over previous
import functools

import jax
import jax.numpy as jnp
from jax import lax
from jax.experimental import pallas as pl
from jax.experimental.pallas import tpu as pltpu
from jax.experimental.pallas import tpu_sc as plsc

_K = 64
_M = 128
_N = 32768
_L = 16
_NVEC = _N // _L
_NC = 2
_NW = 32
_ROWS_PER_W = _M // _NW
_B25 = 0x40200000
_B10 = 0x3F800000


def _splat_total(acc):
    return plsc.cummax(lax.rev(plsc.cumsum(acc), (0,)))


def _splat_max(acc):
    return plsc.cummax(lax.rev(plsc.cummax(acc), (0,)))


def _splat_to_scalar(s, nbits=16):
    out = jnp.int32(0)
    for b in range(nbits):
        bit = jnp.any((s & (1 << b)) != 0)
        out = out + jnp.where(bit, jnp.int32(1 << b), jnp.int32(0))
    return out


def _sc_body(x_hbm, o_hbm, row_v, cix_v, stage_v, vid_v, sem_in, sem_out):
    wid = lax.axis_index("s") * _NC + lax.axis_index("c")
    lanes = lax.iota(jnp.int32, _L)
    one = jnp.ones((_L,), jnp.int32)
    zero = jnp.zeros((_L,), jnp.int32)
    zero16f = jnp.zeros((_L,), jnp.float32)
    kvec = jnp.full((_L,), _K, jnp.int32)
    sixteen = jnp.full((_L,), _L, jnp.int32)
    lane0 = lanes < one
    ninf = jnp.full((_L,), -jnp.inf, jnp.float32)

    def zero_body(i, c):
        stage_v[pl.ds(i * _L, _L)] = zero16f
        return c

    lax.fori_loop(0, _NVEC, zero_body, 0, unroll=8)

    def vz_body(i, c):
        vid_v[pl.ds(i * _L, _L)] = zero
        return c

    lax.fori_loop(0, _NVEC // _L, vz_body, 0, unroll=8)

    pltpu.async_copy(x_hbm.at[wid * _ROWS_PER_W], row_v.at[pl.ds(0, _N)],
                     sem_in)

    def row_body(r, carry):
        nc2_prev, nvec2_prev = carry
        row = wid * _ROWS_PER_W + r
        pltpu.make_async_copy(x_hbm.at[row], row_v.at[pl.ds(0, _N)],
                              sem_in).wait()

        def pass_a(tv, track_max):
            grp = 8

            def body(g, carry):
                nv, mx = carry
                anys = []
                for k in range(grp):
                    i = g * grp + k
                    v = row_v[pl.ds(i * _L, _L)]
                    m = v >= tv
                    anys.append(
                        jnp.minimum(plsc.all_reduce_population_count(m), one))
                    if track_max:
                        mx = jnp.maximum(mx, v)
                pref = [nv]
                for k in range(1, grp):
                    pref.append(pref[k - 1] + anys[k - 1])
                for k in range(grp):
                    i = g * grp + k
                    plsc.store_scatter(vid_v, [pref[k]], zero + i, mask=lane0)
                s1 = [anys[2 * k] + anys[2 * k + 1] for k in range(grp // 2)]
                s2 = [s1[2 * k] + s1[2 * k + 1] for k in range(grp // 4)]
                return nv + s2[0] + s2[1], mx

            return lax.fori_loop(0, _NVEC // grp, body, (zero, ninf))

        def compact2(tv, nvv):
            jmax = (_splat_to_scalar(nvv, 12) + _L - 1) // _L

            def outer(j, off):
                for k in range(_L):
                    slot = j * _L + k
                    vecid = plsc.load_gather(vid_v, [zero + slot])
                    valid = (zero + slot) < nvv
                    ixv = vecid * _L + lanes
                    v = plsc.load_gather(row_v, [ixv])
                    m = (v >= tv) & valid
                    mi = jnp.where(m, one, zero)
                    tgt = off + plsc.cumsum(mi) - mi
                    plsc.store_scatter(cix_v, [tgt], ixv, mask=m)
                    off = off + plsc.all_reduce_population_count(m)
                return off

            return lax.fori_loop(0, jmax, outer, zero)

        t25 = jnp.full((_L,), 2.5, jnp.float32)
        t10 = jnp.full((_L,), 1.0, jnp.float32)
        nv_a, mx = pass_a(t25, True)
        mxbits = plsc.bitcast(_splat_max(mx), jnp.int32)
        hi0 = jnp.maximum(mxbits + one, one)

        def wait_prev(_):
            pltpu.make_async_copy(stage_v, o_hbm.at[row - 1], sem_out).wait()
            return 0

        lax.cond(r > 0, wait_prev, lambda _: 0, 0)

        def rz_body(i, c2):
            valid = (lanes + i * _L) < nc2_prev
            ix = cix_v[pl.ds(i * _L, _L)]
            plsc.store_scatter(stage_v, [ix], zero16f, mask=valid)
            return c2

        lax.fori_loop(0, nvec2_prev, rz_body, 0)

        nc2_a = compact2(t25, nv_a)

        def tier_a(_):
            return nc2_a, jnp.full((_L,), _B25, jnp.int32)

        def tier_bc(_):
            pltpu.sync_copy(x_hbm.at[row], row_v.at[pl.ds(0, _N)])
            nv_b, _ = pass_a(t10, False)
            nc2_b = compact2(t10, nv_b)

            def tier_b(_):
                return nc2_b, jnp.full((_L,), _B10, jnp.int32)

            def tier_c(_):
                pltpu.sync_copy(x_hbm.at[row], row_v.at[pl.ds(0, _N)])
                nv_c, _ = pass_a(ninf, False)
                nc2_c = compact2(ninf, nv_c)
                return nc2_c, zero

            return lax.cond(jnp.any(nc2_b >= kvec), tier_b, tier_c, 0)

        nc2, lo0 = lax.cond(jnp.any(nc2_a >= kvec), tier_a, tier_bc, 0)
        ncand = _splat_to_scalar(nc2)
        nvec2 = (ncand + _L - 1) // _L

        def count_ge(t):
            def b(i, acc):
                valid = (lanes + i * _L) < nc2
                ix = cix_v[pl.ds(i * _L, _L)]
                v = plsc.load_gather(row_v, [ix], mask=valid)
                u = plsc.bitcast(v, jnp.int32)
                return acc + jnp.where((u >= t) & valid, one, zero)

            return _splat_total(lax.fori_loop(0, nvec2, b, zero))

        def bis_cond(lohi):
            lo, hi = lohi
            return jnp.any((hi - lo) > one)

        def bis_body(lohi):
            lo, hi = lohi
            d = hi - lo
            m1 = lo + lax.shift_right_logical(d, 2)
            m2 = lo + lax.shift_right_logical(d, 1)
            m3 = m2 + lax.shift_right_logical(d, 2)

            def b(i, accs):
                a1, a2, a3 = accs
                valid = (lanes + i * _L) < nc2
                ix = cix_v[pl.ds(i * _L, _L)]
                v = plsc.load_gather(row_v, [ix], mask=valid)
                u = plsc.bitcast(v, jnp.int32)
                a1 = a1 + jnp.where((u >= m1) & valid, one, zero)
                a2 = a2 + jnp.where((u >= m2) & valid, one, zero)
                a3 = a3 + jnp.where((u >= m3) & valid, one, zero)
                return a1, a2, a3

            a1, a2, a3 = lax.fori_loop(0, nvec2, b, (zero, zero, zero))
            p1 = _splat_total(a1) >= kvec
            p2 = _splat_total(a2) >= kvec
            p3 = _splat_total(a3) >= kvec
            nlo = jnp.where(p3, m3, jnp.where(p2, m2, jnp.where(p1, m1, lo)))
            nhi = jnp.where(p3, hi, jnp.where(p2, m3, jnp.where(p1, m2, m1)))
            return nlo, nhi

        lo, _ = lax.while_loop(bis_cond, bis_body, (lo0, hi0))
        need = kvec - count_ge(lo + one)

        def sel_body(i, base):
            valid = (lanes + i * _L) < nc2
            ix = cix_v[pl.ds(i * _L, _L)]
            v = plsc.load_gather(row_v, [ix], mask=valid)
            u = plsc.bitcast(v, jnp.int32)
            gt = (u > lo) & valid
            eq = (u == lo) & valid
            pr = plsc.cumsum(jnp.where(eq, one, zero)) + base
            sel = gt | (eq & (pr <= need))
            plsc.store_scatter(stage_v, [ix], jnp.maximum(v, 0.0), mask=sel)
            return base + plsc.all_reduce_population_count(eq)

        lax.fori_loop(0, nvec2, sel_body, zero)

        pltpu.async_copy(stage_v, o_hbm.at[row], sem_out)

        def start_next(_):
            pltpu.async_copy(x_hbm.at[row + 1], row_v.at[pl.ds(0, _N)],
                             sem_in)
            return 0

        lax.cond(r < _ROWS_PER_W - 1, start_next, lambda _: 0, 0)
        return nc2, nvec2

    lax.fori_loop(0, _ROWS_PER_W, row_body, (zero, 0))
    pltpu.make_async_copy(
        stage_v, o_hbm.at[wid * _ROWS_PER_W + _ROWS_PER_W - 1], sem_out
    ).wait()


@functools.partial(
    pl.kernel,
    out_type=jax.ShapeDtypeStruct((_M, _N), jnp.float32),
    mesh=plsc.VectorSubcoreMesh(core_axis_name="c", subcore_axis_name="s"),
    scratch_types=[
        pltpu.VMEM((_N + _L,), jnp.float32),
        pltpu.VMEM((_N,), jnp.int32),
        pltpu.VMEM((_N,), jnp.float32),
        pltpu.VMEM((_NVEC,), jnp.int32),
        pltpu.SemaphoreType.DMA,
        pltpu.SemaphoreType.DMA,
    ],
    compiler_params=pltpu.CompilerParams(needs_layout_passes=False),
)
def _sc_topk(x_hbm, o_hbm, row_v, cix_v, stage_v, vid_v, sem_in, sem_out):
    _sc_body(x_hbm, o_hbm, row_v, cix_v, stage_v, vid_v, sem_in, sem_out)


def kernel(x):
    return _sc_topk(x)
